# baseline (speedup 1.0000x reference)
import functools

import jax
import jax.numpy as jnp
from jax import lax
from jax.experimental import pallas as pl
from jax.experimental.pallas import tpu as pltpu
from jax.experimental.pallas import tpu_sc as plsc

NUM_ROWS = 5
EMB = 64
BATCH = 16384
SEQ = 200
FUSE = 4
FTAB = NUM_ROWS ** FUSE
FEMB = FUSE * EMB
PTOTAL = BATCH * SEQ // FUSE
NW = 32
PER_W = PTOTAL // NW
NSLOT = 8
CHUNK = 40
STEPS = PER_W // CHUNK


def _sc_embed(idx_hbm, table_hbm, out_hbm, *refs):
    idx_vs = refs[0:NSLOT]
    rows_vs = refs[NSLOT:2 * NSLOT]
    g_sems = refs[2 * NSLOT:3 * NSLOT]
    o_sems = refs[3 * NSLOT:4 * NSLOT]

    wid = lax.axis_index("s") * 2 + lax.axis_index("c")
    base = wid * PER_W

    def prep(j, b):
        pltpu.sync_copy(idx_hbm.at[pl.ds(base + j * CHUNK, CHUNK)],
                        idx_vs[b])
        pltpu.async_copy(table_hbm.at[idx_vs[b]], rows_vs[b], g_sems[b])

    def wait_write(b):
        pltpu.make_async_copy(rows_vs[b],
                              out_hbm.at[pl.ds(base, CHUNK)],
                              o_sems[b]).wait()

    for b in range(NSLOT - 1):
        prep(b, b)

    def body(g, carry):
        for b in range(NSLOT):
            j = NSLOT * g + b
            fb = (b - 1) % NSLOT
            fire_ok = j + NSLOT - 1 < STEPS
            pl.when(jnp.logical_and(j >= 1, fire_ok))(
                lambda fb=fb: wait_write(fb))
            pl.when(fire_ok)(lambda j=j, fb=fb: prep(j + NSLOT - 1, fb))
            pltpu.make_async_copy(table_hbm.at[idx_vs[b]],
                                  rows_vs[b], g_sems[b]).wait()
            pltpu.async_copy(rows_vs[b],
                             out_hbm.at[pl.ds(base + j * CHUNK, CHUNK)],
                             o_sems[b])
        return carry

    lax.fori_loop(0, STEPS // NSLOT, body, 0)
    for b in range(NSLOT):
        wait_write(b)


_sc_call = functools.partial(
    pl.kernel,
    out_type=jax.ShapeDtypeStruct((PTOTAL, FEMB), jnp.float32),
    mesh=plsc.VectorSubcoreMesh(core_axis_name="c", subcore_axis_name="s"),
    scratch_types=(
        [pltpu.VMEM((CHUNK,), jnp.int32) for _ in range(NSLOT)]
        + [pltpu.VMEM((CHUNK, FEMB), jnp.float32) for _ in range(NSLOT)]
        + [pltpu.SemaphoreType.DMA for _ in range(2 * NSLOT)]
    ),
)(_sc_embed)


@jax.jit
def kernel(inputs, table):
    flat = inputs.reshape(PTOTAL, FUSE)
    fidx = flat[:, 0]
    for k in range(1, FUSE):
        fidx = fidx * NUM_ROWS + flat[:, k]
    parts = [
        jnp.tile(jnp.repeat(table, NUM_ROWS ** (FUSE - 1 - k), axis=0),
                 (NUM_ROWS ** k, 1))
        for k in range(FUSE)
    ]
    ftab = jnp.concatenate(parts, axis=1)
    out = _sc_call(fidx, ftab)
    return out.reshape(BATCH, SEQ, EMB)


# speedup vs baseline: 1.0029x; 1.0029x over previous
import functools

import jax
import jax.numpy as jnp
from jax import lax
from jax.experimental import pallas as pl
from jax.experimental.pallas import tpu as pltpu
from jax.experimental.pallas import tpu_sc as plsc

NUM_ROWS = 5
EMB = 64
BATCH = 16384
SEQ = 200
FUSE = 4
FTAB = NUM_ROWS ** FUSE
FEMB = FUSE * EMB
PTOTAL = BATCH * SEQ // FUSE
NW = 32
PER_W = PTOTAL // NW
NSLOT = 4
CHUNK = 80
STEPS = PER_W // CHUNK


def _sc_embed(idx_hbm, table_hbm, out_hbm, *refs):
    idx_vs = refs[0:NSLOT]
    rows_vs = refs[NSLOT:2 * NSLOT]
    g_sems = refs[2 * NSLOT:3 * NSLOT]
    o_sems = refs[3 * NSLOT:4 * NSLOT]

    wid = lax.axis_index("c") * 16 + lax.axis_index("s")
    base = wid * PER_W

    def prep(j, b):
        pltpu.sync_copy(idx_hbm.at[pl.ds(base + j * CHUNK, CHUNK)],
                        idx_vs[b])
        pltpu.async_copy(table_hbm.at[idx_vs[b]], rows_vs[b], g_sems[b])

    def wait_write(b):
        pltpu.make_async_copy(rows_vs[b],
                              out_hbm.at[pl.ds(base, CHUNK)],
                              o_sems[b]).wait()

    for b in range(NSLOT - 1):
        prep(b, b)

    def body(g, carry):
        for b in range(NSLOT):
            j = NSLOT * g + b
            fb = (b - 1) % NSLOT
            fire_ok = j + NSLOT - 1 < STEPS
            pl.when(jnp.logical_and(j >= 1, fire_ok))(
                lambda fb=fb: wait_write(fb))
            pl.when(fire_ok)(lambda j=j, fb=fb: prep(j + NSLOT - 1, fb))
            pltpu.make_async_copy(table_hbm.at[idx_vs[b]],
                                  rows_vs[b], g_sems[b]).wait()
            pltpu.async_copy(rows_vs[b],
                             out_hbm.at[pl.ds(base + j * CHUNK, CHUNK)],
                             o_sems[b])
        return carry

    lax.fori_loop(0, STEPS // NSLOT, body, 0)
    for b in range(NSLOT):
        wait_write(b)


_sc_call = functools.partial(
    pl.kernel,
    out_type=jax.ShapeDtypeStruct((PTOTAL, FEMB), jnp.float32),
    mesh=plsc.VectorSubcoreMesh(core_axis_name="c", subcore_axis_name="s"),
    scratch_types=(
        [pltpu.VMEM((CHUNK,), jnp.int32) for _ in range(NSLOT)]
        + [pltpu.VMEM((CHUNK, FEMB), jnp.float32) for _ in range(NSLOT)]
        + [pltpu.SemaphoreType.DMA for _ in range(2 * NSLOT)]
    ),
)(_sc_embed)


@jax.jit
def kernel(inputs, table):
    flat = inputs.reshape(PTOTAL, FUSE)
    fidx = flat[:, 0]
    for k in range(1, FUSE):
        fidx = fidx * NUM_ROWS + flat[:, k]
    parts = [
        jnp.tile(jnp.repeat(table, NUM_ROWS ** (FUSE - 1 - k), axis=0),
                 (NUM_ROWS ** k, 1))
        for k in range(FUSE)
    ]
    ftab = jnp.concatenate(parts, axis=1)
    out = _sc_call(fidx, ftab)
    return out.reshape(BATCH, SEQ, EMB)


# speedup vs baseline: 1.0065x; 1.0036x over previous
import functools

import jax
import jax.numpy as jnp
from jax import lax
from jax.experimental import pallas as pl
from jax.experimental.pallas import tpu as pltpu
from jax.experimental.pallas import tpu_sc as plsc

NUM_ROWS = 5
EMB = 64
BATCH = 16384
SEQ = 200
FUSE = 4
FTAB = NUM_ROWS ** FUSE
FEMB = FUSE * EMB
PTOTAL = BATCH * SEQ // FUSE
NW = 32
PER_W = PTOTAL // NW
NSLOT = 2
CHUNK = 128
STEPS = PER_W // CHUNK
IDROWS = PTOTAL // CHUNK


def _sc_embed(idx_hbm, table_hbm, out_hbm, *refs):
    idx_vs = refs[0:NSLOT]
    rows_vs = refs[NSLOT:2 * NSLOT]
    g_sems = refs[2 * NSLOT:3 * NSLOT]
    o_sems = refs[3 * NSLOT:4 * NSLOT]

    wid = lax.axis_index("c") * 16 + lax.axis_index("s")
    base = wid * PER_W

    base_row = wid * STEPS

    def prep(j, b):
        pltpu.sync_copy(idx_hbm.at[base_row + j], idx_vs[b])
        pltpu.async_copy(table_hbm.at[idx_vs[b]], rows_vs[b], g_sems[b])

    def wait_write(b):
        pltpu.make_async_copy(rows_vs[b],
                              out_hbm.at[pl.ds(base, CHUNK)],
                              o_sems[b]).wait()

    for b in range(NSLOT - 1):
        prep(b, b)

    def body(g, carry):
        for b in range(NSLOT):
            j = NSLOT * g + b
            fb = (b - 1) % NSLOT
            fire_ok = j + NSLOT - 1 < STEPS
            pl.when(jnp.logical_and(j >= 1, fire_ok))(
                lambda fb=fb: wait_write(fb))
            pl.when(fire_ok)(lambda j=j, fb=fb: prep(j + NSLOT - 1, fb))
            pltpu.make_async_copy(table_hbm.at[idx_vs[b]],
                                  rows_vs[b], g_sems[b]).wait()
            pltpu.async_copy(rows_vs[b],
                             out_hbm.at[pl.ds(base + j * CHUNK, CHUNK)],
                             o_sems[b])
        return carry

    lax.fori_loop(0, STEPS // NSLOT, body, 0)
    for b in range(NSLOT):
        wait_write(b)


_sc_call = functools.partial(
    pl.kernel,
    out_type=jax.ShapeDtypeStruct((PTOTAL, FEMB), jnp.float32),
    mesh=plsc.VectorSubcoreMesh(core_axis_name="c", subcore_axis_name="s"),
    scratch_types=(
        [pltpu.VMEM((CHUNK,), jnp.int32) for _ in range(NSLOT)]
        + [pltpu.VMEM((CHUNK, FEMB), jnp.float32) for _ in range(NSLOT)]
        + [pltpu.SemaphoreType.DMA for _ in range(2 * NSLOT)]
    ),
)(_sc_embed)


@jax.jit
def kernel(inputs, table):
    flat = inputs.reshape(PTOTAL, FUSE)
    fidx = flat[:, 0]
    for k in range(1, FUSE):
        fidx = fidx * NUM_ROWS + flat[:, k]
    fidx = fidx.reshape(IDROWS, CHUNK)
    parts = [
        jnp.tile(jnp.repeat(table, NUM_ROWS ** (FUSE - 1 - k), axis=0),
                 (NUM_ROWS ** k, 1))
        for k in range(FUSE)
    ]
    ftab = jnp.concatenate(parts, axis=1)
    out = _sc_call(fidx, ftab)
    return out.reshape(BATCH, SEQ, EMB)


# speedup vs baseline: 1.0081x; 1.0016x over previous
import functools

import jax
import jax.numpy as jnp
from jax import lax
from jax.experimental import pallas as pl
from jax.experimental.pallas import tpu as pltpu
from jax.experimental.pallas import tpu_sc as plsc

NUM_ROWS = 5
EMB = 64
BATCH = 16384
SEQ = 200
FUSE = 4
FEMB = FUSE * EMB
PTOTAL = BATCH * SEQ // FUSE
NW = 32
PER_W = PTOTAL // NW
NSLOT = 2
CHUNK = 128
STEPS = PER_W // CHUNK
IDROWS = PTOTAL // CHUNK


def _sc_embed(idx_hbm, table_hbm, out_hbm, *refs):
    idx_vs = refs[0:NSLOT]
    rows_vs = refs[NSLOT:2 * NSLOT]
    g_sems = refs[2 * NSLOT:3 * NSLOT]
    o_sems = refs[3 * NSLOT:4 * NSLOT]

    wid = lax.axis_index("c") * 16 + lax.axis_index("s")
    base = wid * PER_W

    base_row = wid * STEPS

    def prep(j, b):
        pltpu.sync_copy(idx_hbm.at[base_row + j], idx_vs[b])
        pltpu.async_copy(table_hbm.at[idx_vs[b]], rows_vs[b], g_sems[b])

    def wait_write(b):
        pltpu.make_async_copy(rows_vs[b],
                              out_hbm.at[pl.ds(base, CHUNK)],
                              o_sems[b]).wait()

    for b in range(NSLOT - 1):
        prep(b, b)

    def body(g, carry):
        for b in range(NSLOT):
            j = NSLOT * g + b
            fb = (b - 1) % NSLOT
            fire_ok = j + NSLOT - 1 < STEPS
            pl.when(jnp.logical_and(j >= 1, fire_ok))(
                lambda fb=fb: wait_write(fb))
            pl.when(fire_ok)(lambda j=j, fb=fb: prep(j + NSLOT - 1, fb))
            pltpu.make_async_copy(table_hbm.at[idx_vs[b]],
                                  rows_vs[b], g_sems[b]).wait()
            pltpu.async_copy(rows_vs[b],
                             out_hbm.at[pl.ds(base + j * CHUNK, CHUNK)],
                             o_sems[b])
        return carry

    lax.fori_loop(0, STEPS // NSLOT, body, 0)
    for b in range(NSLOT):
        wait_write(b)


_sc_call = functools.partial(
    pl.kernel,
    out_type=jax.ShapeDtypeStruct((PTOTAL, FEMB), jnp.float32),
    mesh=plsc.VectorSubcoreMesh(core_axis_name="c", subcore_axis_name="s"),
    scratch_types=(
        [pltpu.VMEM((CHUNK,), jnp.int32) for _ in range(NSLOT)]
        + [pltpu.VMEM((CHUNK, FEMB), jnp.float32) for _ in range(NSLOT)]
        + [pltpu.SemaphoreType.DMA for _ in range(2 * NSLOT)]
    ),
)(_sc_embed)


@jax.jit
def kernel(inputs, table):
    flat = inputs.reshape(PTOTAL, FUSE)
    fidx = flat[:, 0]
    for k in range(1, FUSE):
        fidx = fidx * NUM_ROWS + flat[:, k]
    fidx = fidx.reshape(IDROWS, CHUNK)
    parts = [
        jnp.tile(jnp.repeat(table, NUM_ROWS ** (FUSE - 1 - k), axis=0),
                 (NUM_ROWS ** k, 1))
        for k in range(FUSE)
    ]
    ftab = jnp.concatenate(parts, axis=1)
    out = _sc_call(fidx, ftab)
    return out.reshape(BATCH, SEQ, EMB)
